# Initial kernel scaffold; baseline (speedup 1.0000x reference)
import jax
import jax.numpy as jnp
from jax.experimental import pallas as pl


def kernel(x, edge_attr, edge_index, enc_We, enc_be, enc_Wn, enc_bn, proc_We, proc_be, proc_Wn, proc_bn, dec_We, dec_be, dec_Wn, dec_bn):
    raise NotImplementedError("write your pallas kernel here")


# speedup vs baseline: 2.9006x; 2.9006x over previous
import functools

import jax
import jax.numpy as jnp
from jax import lax
from jax.experimental import pallas as pl
from jax.experimental.pallas import tpu as pltpu
from jax.experimental.pallas import tpu_sc as plsc

_N = 10000
_E = 320000
_L = 32
_NC = 2
_NS = 16
_NW = _NC * _NS
_CHUNK = 128
_NCHUNKS = _E // _CHUNK
_RPS = (_N // _NS) // 8 * 8
_BE = 16000

_f32 = jnp.float32


def _make_sc_stage(write_h):
    mesh = plsc.VectorSubcoreMesh(core_axis_name="c", subcore_axis_name="s",
                                  num_cores=_NC, num_subcores=_NS)
    out_type = [jax.ShapeDtypeStruct((_NC, _N, _L), _f32)]
    if write_h:
        out_type = [jax.ShapeDtypeStruct((_E, _L), _f32)] + out_type
    scratch_types = [
        pltpu.VMEM((_CHUNK,), jnp.int32),
        pltpu.VMEM((_CHUNK,), jnp.int32),
        pltpu.VMEM((_CHUNK, _L), _f32),
        pltpu.VMEM((_CHUNK, _L), _f32),
        pltpu.VMEM((_CHUNK, _L), _f32),
        pltpu.VMEM_SHARED((_N, _L), _f32),
        pltpu.SemaphoreType.DMA,
    ]

    def body(sidx_hbm, ridx_hbm, loc_hbm, tabs_hbm, tabr_hbm, zero_hbm, *rest):
        if write_h:
            h_hbm, aggp_hbm = rest[0], rest[1]
            sidx_v, ridx_v, loc_v, gs_v, gr_v, agg_sh, sem = rest[2:]
        else:
            aggp_hbm = rest[0]
            sidx_v, ridx_v, loc_v, gs_v, gr_v, agg_sh, sem = rest[1:]
        cid = lax.axis_index("c")
        sid = lax.axis_index("s")
        wid = sid * _NC + cid

        @pl.when(sid == 0)
        def _():
            pltpu.sync_copy(zero_hbm, agg_sh)

        plsc.subcore_barrier()

        nch = _NCHUNKS // _NW + jnp.where(wid < _NCHUNKS % _NW, 1, 0)

        def chunk_body(j, carry):
            base = (wid + j * _NW) * _CHUNK
            pltpu.sync_copy(sidx_hbm.at[pl.ds(base, _CHUNK)], sidx_v)
            pltpu.sync_copy(ridx_hbm.at[pl.ds(base, _CHUNK)], ridx_v)
            pltpu.sync_copy(loc_hbm.at[pl.ds(base, _CHUNK)], loc_v)
            pltpu.async_copy(tabs_hbm.at[sidx_v], gs_v, sem).wait()
            pltpu.async_copy(tabr_hbm.at[ridx_v], gr_v, sem).wait()

            def vrow(i, c2):
                v0 = loc_v[i, pl.ds(0, 16)] + gs_v[i, pl.ds(0, 16)] + gr_v[i, pl.ds(0, 16)]
                loc_v[i, pl.ds(0, 16)] = jnp.maximum(v0, 0.0)
                v1 = loc_v[i, pl.ds(16, 16)] + gs_v[i, pl.ds(16, 16)] + gr_v[i, pl.ds(16, 16)]
                loc_v[i, pl.ds(16, 16)] = jnp.maximum(v1, 0.0)
                return c2

            lax.fori_loop(0, _CHUNK, vrow, 0)
            if write_h:
                pltpu.sync_copy(loc_v, h_hbm.at[pl.ds(base, _CHUNK)])
            pltpu.sync_copy(loc_v, agg_sh.at[ridx_v], add=True)
            return carry

        lax.fori_loop(0, nch, chunk_body, 0)
        plsc.subcore_barrier()
        pltpu.sync_copy(
            agg_sh.at[pl.ds(sid * _RPS, _RPS)],
            aggp_hbm.at[cid, pl.ds(sid * _RPS, _RPS)],
        )

        @pl.when(sid == 0)
        def _():
            pltpu.sync_copy(
                agg_sh.at[pl.ds(_NS * _RPS, _N - _NS * _RPS)],
                aggp_hbm.at[cid, pl.ds(_NS * _RPS, _N - _NS * _RPS)],
            )

    return pl.kernel(body, out_type=tuple(out_type) if write_h else out_type[0],
                     mesh=mesh, scratch_types=scratch_types,
                     compiler_params=pltpu.CompilerParams(use_tc_tiling_on_sc=False))


_sc_stage_h = _make_sc_stage(True)
_sc_stage_last = _make_sc_stage(False)


def _dot(a, b):
    return jnp.dot(a, b, preferred_element_type=_f32)


def _edge_call(nk, in_widths, w_shapes, n_out):
    grid = (_E // _BE,)
    in_specs = [pl.BlockSpec((_BE, w), lambda i: (i, 0)) for w in in_widths]
    in_specs += [pl.BlockSpec(shp, lambda i: (0, 0)) for shp in w_shapes]
    out_specs = [pl.BlockSpec((_BE, _L), lambda i: (i, 0))] * n_out
    out_shape = [jax.ShapeDtypeStruct((_E, _L), _f32)] * n_out
    if n_out == 1:
        out_specs, out_shape = out_specs[0], out_shape[0]
    return pl.pallas_call(nk, grid=grid, in_specs=in_specs,
                          out_specs=out_specs, out_shape=out_shape)


def _edge1_k(ea_ref, w_ref, b_ref, o_ref):
    o_ref[...] = _dot(ea_ref[...], w_ref[...]) + b_ref[...]


def _edge2_k(h_ref, wl_ref, w2_ref, b_ref, o1_ref, o2_ref):
    h = h_ref[...]
    o1_ref[...] = _dot(h, wl_ref[...]) + b_ref[...]
    o2_ref[...] = _dot(h, w2_ref[...])


def _edge3_k(h_ref, pb_ref, w_ref, b_ref, o_ref):
    o_ref[...] = _dot(h_ref[...], w_ref[...]) + pb_ref[...] + b_ref[...]


def _edge4_k(h_ref, w_ref, b_ref, o_ref):
    o_ref[...] = _dot(h_ref[...], w_ref[...]) + b_ref[...]


_edge1 = _edge_call(_edge1_k, [16], [(16, _L), (1, _L)], 1)
_edge2 = _edge_call(_edge2_k, [_L], [(_L, _L), (_L, _L), (1, _L)], 2)
_edge3 = _edge_call(_edge3_k, [_L, _L], [(_L, _L), (1, _L)], 1)
_edge4 = _edge_call(_edge4_k, [_L], [(_L, _L), (1, _L)], 1)


def _node_call(nk, n_in, n_out):
    out_shape = [jax.ShapeDtypeStruct((_N, _L), _f32)] * n_out
    if n_out == 1:
        out_shape = out_shape[0]
    return pl.pallas_call(nk, out_shape=out_shape)


def _node1_k(x_ref, ws_ref, wr_ref, wnx_ref, a_ref, b_ref, nx_ref):
    xx = x_ref[...]
    a_ref[...] = _dot(xx, ws_ref[...])
    b_ref[...] = _dot(xx, wr_ref[...])
    nx_ref[...] = _dot(xx, wnx_ref[...])


def _node2_k(aggp_ref, nx_ref, wna_ref, bn_ref, ws_ref, wr_ref, wn_ref,
             hx_ref, p2s_ref, p2r_ref, n2_ref):
    agg = aggp_ref[0] + aggp_ref[1]
    hx = jnp.maximum(nx_ref[...] + _dot(agg, wna_ref[...]) + bn_ref[...], 0.0)
    hx_ref[...] = hx
    p2s_ref[...] = _dot(hx, ws_ref[...])
    p2r_ref[...] = _dot(hx, wr_ref[...])
    n2_ref[...] = _dot(hx, wn_ref[...])


def _node3_k(aggp_ref, n2_ref, hx_ref, wna_ref, bn_ref,
             ws1_ref, ws2_ref, wr1_ref, wr2_ref, wn1_ref, wn2_ref,
             p3s_ref, p3r_ref, n3_ref):
    agg = aggp_ref[0] + aggp_ref[1]
    cx2 = jnp.maximum(n2_ref[...] + _dot(agg, wna_ref[...]) + bn_ref[...], 0.0)
    hx = hx_ref[...]
    p3s_ref[...] = _dot(cx2, ws1_ref[...]) + _dot(hx, ws2_ref[...])
    p3r_ref[...] = _dot(cx2, wr1_ref[...]) + _dot(hx, wr2_ref[...])
    n3_ref[...] = _dot(cx2, wn1_ref[...]) + _dot(hx, wn2_ref[...])


def _node4_k(aggp_ref, n3_ref, wna_ref, bn_ref, ws_ref, wr_ref, wnx_ref,
             p4s_ref, p4r_ref, n4_ref):
    agg = aggp_ref[0] + aggp_ref[1]
    cx3 = jnp.maximum(n3_ref[...] + _dot(agg, wna_ref[...]) + bn_ref[...], 0.0)
    p4s_ref[...] = _dot(cx3, ws_ref[...])
    p4r_ref[...] = _dot(cx3, wr_ref[...])
    n4_ref[...] = _dot(cx3, wnx_ref[...])


def _node5_k(aggp_ref, n4_ref, wna_ref, bn_ref, o_ref):
    agg = aggp_ref[0] + aggp_ref[1]
    o_ref[...] = jnp.maximum(n4_ref[...] + _dot(agg, wna_ref[...]) + bn_ref[...], 0.0)


_node1 = _node_call(_node1_k, 4, 3)
_node2 = _node_call(_node2_k, 7, 4)
_node3 = _node_call(_node3_k, 11, 3)
_node4 = _node_call(_node4_k, 7, 3)
_node5 = _node_call(_node5_k, 4, 1)


def kernel(x, edge_attr, edge_index,
           enc_We, enc_be, enc_Wn, enc_bn,
           proc_We, proc_be, proc_Wn, proc_bn,
           dec_We, dec_be, dec_Wn, dec_bn):
    s = edge_index[0]
    r = edge_index[1]

    we_e, we_s, we_r = enc_We[:16], enc_We[16:144], enc_We[144:272]
    wn_x, wn_a = enc_Wn[:128], enc_Wn[128:160]
    pwe1, pwe2 = proc_We[0:32], proc_We[32:64]
    pwes1, pwes2 = proc_We[64:96], proc_We[96:128]
    pwer1, pwer2 = proc_We[128:160], proc_We[160:192]
    pwn1, pwn2, pwna = proc_Wn[0:32], proc_Wn[32:64], proc_Wn[64:96]
    dwe_e, dwe_s, dwe_r = dec_We[0:32], dec_We[32:64], dec_We[64:96]
    dwn_x, dwn_a = dec_Wn[0:32], dec_Wn[32:64]

    ebe = enc_be.reshape(1, _L)
    ebn = enc_bn.reshape(1, _L)
    pbe = proc_be.reshape(1, _L)
    pbn = proc_bn.reshape(1, _L)
    dbe = dec_be.reshape(1, _L)
    dbn = dec_bn.reshape(1, _L)

    zero = jnp.zeros((_N, _L), _f32)

    loc1 = _edge1(edge_attr, we_e, ebe)
    a1, b1, nx = _node1(x, we_s, we_r, wn_x)
    h1, aggp1 = _sc_stage_h(s, r, loc1, a1, b1, zero)

    hx, p2s, p2r, n2 = _node2(aggp1, nx, wn_a, ebn,
                              pwes1 + pwes2, pwer1 + pwer2, pwn1 + pwn2)
    loc2, partb = _edge2(h1, pwe1 + pwe2, pwe2, pbe)
    h2, aggp2 = _sc_stage_h(s, r, loc2, p2s, p2r, zero)

    p3s, p3r, n3 = _node3(aggp2, n2, hx, pwna, pbn,
                          pwes1, pwes2, pwer1, pwer2, pwn1, pwn2)
    loc3 = _edge3(h2, partb, pwe1, pbe)
    h3, aggp3 = _sc_stage_h(s, r, loc3, p3s, p3r, zero)

    p4s, p4r, n4 = _node4(aggp3, n3, pwna, pbn, dwe_s, dwe_r, dwn_x)
    loc4 = _edge4(h3, dwe_e, dbe)
    aggp4 = _sc_stage_last(s, r, loc4, p4s, p4r, zero)

    return _node5(aggp4, n4, dwn_a, dbn)


# speedup vs baseline: 4.3276x; 1.4919x over previous
import functools

import jax
import jax.numpy as jnp
from jax import lax
from jax.experimental import pallas as pl
from jax.experimental.pallas import tpu as pltpu
from jax.experimental.pallas import tpu_sc as plsc

_N = 10000
_E = 320000
_L = 32
_NC = 2
_NS = 16
_NW = _NC * _NS
_CHUNK = 128
_NCHUNKS = _E // _CHUNK
_RPS = (_N // _NS) // 8 * 8
_BE = 16000
_DEPTH = 6
_NJ = (_NCHUNKS // _NW) // _DEPTH * _DEPTH
_NIT = _NJ // _DEPTH
_NTAIL = _NCHUNKS - _NJ * _NW

_f32 = jnp.float32


def _make_sc_stage(write_h):
    mesh = plsc.VectorSubcoreMesh(core_axis_name="c", subcore_axis_name="s",
                                  num_cores=_NC, num_subcores=_NS)
    out_type = [jax.ShapeDtypeStruct((_NC, _N, _L), _f32)]
    if write_h:
        out_type = [jax.ShapeDtypeStruct((_E, _L), _f32)] + out_type
    slot_scratch = [
        pltpu.VMEM((2, _CHUNK), jnp.int32),
        pltpu.VMEM((_CHUNK, _L), _f32),
        pltpu.VMEM((_CHUNK, _L), _f32),
        pltpu.VMEM((_CHUNK, _L), _f32),
    ]
    scratch_types = (slot_scratch * _DEPTH
                     + [pltpu.VMEM_SHARED((_N, _L), _f32)]
                     + [pltpu.SemaphoreType.DMA] * (_DEPTH + 2))

    def body(sr_hbm, loc_hbm, tabs_hbm, tabr_hbm, zero_hbm, *rest):
        if write_h:
            h_hbm, aggp_hbm = rest[0], rest[1]
            rest = rest[2:]
        else:
            aggp_hbm = rest[0]
            rest = rest[1:]
        slots = [rest[4 * k:4 * k + 4] for k in range(_DEPTH)]
        agg_sh = rest[4 * _DEPTH]
        semg = rest[4 * _DEPTH + 1:4 * _DEPTH + 1 + _DEPTH]
        semi = rest[4 * _DEPTH + 1 + _DEPTH]
        semw = rest[4 * _DEPTH + 2 + _DEPTH]
        cid = lax.axis_index("c")
        sid = lax.axis_index("s")
        wid = sid * _NC + cid

        @pl.when(sid == 0)
        def _():
            pltpu.sync_copy(zero_hbm, agg_sh)

        plsc.subcore_barrier()

        def compute(b):
            lv, gs, gr = slots[b][1], slots[b][2], slots[b][3]

            @plsc.parallel_loop(0, _CHUNK, unroll=4)
            def _(i):
                v0 = lv[i, pl.ds(0, 16)] + gs[i, pl.ds(0, 16)] + gr[i, pl.ds(0, 16)]
                lv[i, pl.ds(0, 16)] = jnp.maximum(v0, 0.0)
                v1 = lv[i, pl.ds(16, 16)] + gs[i, pl.ds(16, 16)] + gr[i, pl.ds(16, 16)]
                lv[i, pl.ds(16, 16)] = jnp.maximum(v1, 0.0)

        def iteration(it, carry):
            c0 = wid + it * (_DEPTH * _NW)
            idx_waits = []
            for b in range(_DEPTH):
                d = pltpu.async_copy(sr_hbm.at[c0 + b * _NW], slots[b][0], semi)
                idx_waits.append(d)
            for d in idx_waits:
                d.wait()
            data_waits = []
            for b in range(_DEPTH):
                idxv, lv, gs, gr = slots[b]
                base = (c0 + b * _NW) * _CHUNK
                d1 = pltpu.async_copy(loc_hbm.at[pl.ds(base, _CHUNK)], lv, semg[b])
                d2 = pltpu.async_copy(tabs_hbm.at[idxv.at[0]], gs, semg[b])
                d3 = pltpu.async_copy(tabr_hbm.at[idxv.at[1]], gr, semg[b])
                data_waits.append((d1, d2, d3))
            h_waits = []
            for b in range(_DEPTH):
                idxv, lv, gs, gr = slots[b]
                for d in data_waits[b]:
                    d.wait()
                compute(b)
                if write_h:
                    base = (c0 + b * _NW) * _CHUNK
                    h_waits.append(
                        pltpu.async_copy(lv, h_hbm.at[pl.ds(base, _CHUNK)], semw))
                pltpu.sync_copy(lv, agg_sh.at[idxv.at[1]], add=True)
            for d in h_waits:
                d.wait()
            return carry

        lax.fori_loop(0, _NIT, iteration, 0)

        @pl.when(wid < _NTAIL)
        def _():
            ct = _NJ * _NW + wid
            base = ct * _CHUNK
            idxv, lv, gs, gr = slots[0]
            pltpu.sync_copy(sr_hbm.at[ct], idxv)
            pltpu.sync_copy(loc_hbm.at[pl.ds(base, _CHUNK)], lv)
            pltpu.async_copy(tabs_hbm.at[idxv.at[0]], gs, semg[0]).wait()
            pltpu.async_copy(tabr_hbm.at[idxv.at[1]], gr, semg[0]).wait()
            compute(0)
            if write_h:
                pltpu.sync_copy(lv, h_hbm.at[pl.ds(base, _CHUNK)])
            pltpu.sync_copy(lv, agg_sh.at[idxv.at[1]], add=True)

        plsc.subcore_barrier()
        pltpu.sync_copy(
            agg_sh.at[pl.ds(sid * _RPS, _RPS)],
            aggp_hbm.at[cid, pl.ds(sid * _RPS, _RPS)],
        )

        @pl.when(sid == 0)
        def _():
            pltpu.sync_copy(
                agg_sh.at[pl.ds(_NS * _RPS, _N - _NS * _RPS)],
                aggp_hbm.at[cid, pl.ds(_NS * _RPS, _N - _NS * _RPS)],
            )

    return pl.kernel(body, out_type=tuple(out_type) if write_h else out_type[0],
                     mesh=mesh, scratch_types=scratch_types,
                     compiler_params=pltpu.CompilerParams(use_tc_tiling_on_sc=False))


_sc_stage_h = _make_sc_stage(True)
_sc_stage_last = _make_sc_stage(False)


def _dot(a, b):
    return jnp.dot(a, b, preferred_element_type=_f32)


def _edge_call(nk, in_widths, w_shapes, n_out):
    grid = (_E // _BE,)
    in_specs = [pl.BlockSpec((_BE, w), lambda i: (i, 0)) for w in in_widths]
    in_specs += [pl.BlockSpec(shp, lambda i: (0, 0)) for shp in w_shapes]
    out_specs = [pl.BlockSpec((_BE, _L), lambda i: (i, 0))] * n_out
    out_shape = [jax.ShapeDtypeStruct((_E, _L), _f32)] * n_out
    if n_out == 1:
        out_specs, out_shape = out_specs[0], out_shape[0]
    return pl.pallas_call(nk, grid=grid, in_specs=in_specs,
                          out_specs=out_specs, out_shape=out_shape)


def _edge1_k(ea_ref, w_ref, b_ref, o_ref):
    o_ref[...] = _dot(ea_ref[...], w_ref[...]) + b_ref[...]


def _edge2_k(h_ref, wl_ref, w2_ref, b_ref, o1_ref, o2_ref):
    h = h_ref[...]
    o1_ref[...] = _dot(h, wl_ref[...]) + b_ref[...]
    o2_ref[...] = _dot(h, w2_ref[...])


def _edge3_k(h_ref, pb_ref, w_ref, b_ref, o_ref):
    o_ref[...] = _dot(h_ref[...], w_ref[...]) + pb_ref[...] + b_ref[...]


def _edge4_k(h_ref, w_ref, b_ref, o_ref):
    o_ref[...] = _dot(h_ref[...], w_ref[...]) + b_ref[...]


_edge1 = _edge_call(_edge1_k, [16], [(16, _L), (1, _L)], 1)
_edge2 = _edge_call(_edge2_k, [_L], [(_L, _L), (_L, _L), (1, _L)], 2)
_edge3 = _edge_call(_edge3_k, [_L, _L], [(_L, _L), (1, _L)], 1)
_edge4 = _edge_call(_edge4_k, [_L], [(_L, _L), (1, _L)], 1)


def _node_call(nk, n_in, n_out):
    out_shape = [jax.ShapeDtypeStruct((_N, _L), _f32)] * n_out
    if n_out == 1:
        out_shape = out_shape[0]
    return pl.pallas_call(nk, out_shape=out_shape)


def _node1_k(x_ref, ws_ref, wr_ref, wnx_ref, a_ref, b_ref, nx_ref):
    xx = x_ref[...]
    a_ref[...] = _dot(xx, ws_ref[...])
    b_ref[...] = _dot(xx, wr_ref[...])
    nx_ref[...] = _dot(xx, wnx_ref[...])


def _node2_k(aggp_ref, nx_ref, wna_ref, bn_ref, ws_ref, wr_ref, wn_ref,
             hx_ref, p2s_ref, p2r_ref, n2_ref):
    agg = aggp_ref[0] + aggp_ref[1]
    hx = jnp.maximum(nx_ref[...] + _dot(agg, wna_ref[...]) + bn_ref[...], 0.0)
    hx_ref[...] = hx
    p2s_ref[...] = _dot(hx, ws_ref[...])
    p2r_ref[...] = _dot(hx, wr_ref[...])
    n2_ref[...] = _dot(hx, wn_ref[...])


def _node3_k(aggp_ref, n2_ref, hx_ref, wna_ref, bn_ref,
             ws1_ref, ws2_ref, wr1_ref, wr2_ref, wn1_ref, wn2_ref,
             p3s_ref, p3r_ref, n3_ref):
    agg = aggp_ref[0] + aggp_ref[1]
    cx2 = jnp.maximum(n2_ref[...] + _dot(agg, wna_ref[...]) + bn_ref[...], 0.0)
    hx = hx_ref[...]
    p3s_ref[...] = _dot(cx2, ws1_ref[...]) + _dot(hx, ws2_ref[...])
    p3r_ref[...] = _dot(cx2, wr1_ref[...]) + _dot(hx, wr2_ref[...])
    n3_ref[...] = _dot(cx2, wn1_ref[...]) + _dot(hx, wn2_ref[...])


def _node4_k(aggp_ref, n3_ref, wna_ref, bn_ref, ws_ref, wr_ref, wnx_ref,
             p4s_ref, p4r_ref, n4_ref):
    agg = aggp_ref[0] + aggp_ref[1]
    cx3 = jnp.maximum(n3_ref[...] + _dot(agg, wna_ref[...]) + bn_ref[...], 0.0)
    p4s_ref[...] = _dot(cx3, ws_ref[...])
    p4r_ref[...] = _dot(cx3, wr_ref[...])
    n4_ref[...] = _dot(cx3, wnx_ref[...])


def _node5_k(aggp_ref, n4_ref, wna_ref, bn_ref, o_ref):
    agg = aggp_ref[0] + aggp_ref[1]
    o_ref[...] = jnp.maximum(n4_ref[...] + _dot(agg, wna_ref[...]) + bn_ref[...], 0.0)


_node1 = _node_call(_node1_k, 4, 3)
_node2 = _node_call(_node2_k, 7, 4)
_node3 = _node_call(_node3_k, 11, 3)
_node4 = _node_call(_node4_k, 7, 3)
_node5 = _node_call(_node5_k, 4, 1)


def kernel(x, edge_attr, edge_index,
           enc_We, enc_be, enc_Wn, enc_bn,
           proc_We, proc_be, proc_Wn, proc_bn,
           dec_We, dec_be, dec_Wn, dec_bn):
    s = edge_index[0]
    r = edge_index[1]
    sr = jnp.stack([s.reshape(_NCHUNKS, _CHUNK), r.reshape(_NCHUNKS, _CHUNK)],
                   axis=1)

    we_e, we_s, we_r = enc_We[:16], enc_We[16:144], enc_We[144:272]
    wn_x, wn_a = enc_Wn[:128], enc_Wn[128:160]
    pwe1, pwe2 = proc_We[0:32], proc_We[32:64]
    pwes1, pwes2 = proc_We[64:96], proc_We[96:128]
    pwer1, pwer2 = proc_We[128:160], proc_We[160:192]
    pwn1, pwn2, pwna = proc_Wn[0:32], proc_Wn[32:64], proc_Wn[64:96]
    dwe_e, dwe_s, dwe_r = dec_We[0:32], dec_We[32:64], dec_We[64:96]
    dwn_x, dwn_a = dec_Wn[0:32], dec_Wn[32:64]

    ebe = enc_be.reshape(1, _L)
    ebn = enc_bn.reshape(1, _L)
    pbe = proc_be.reshape(1, _L)
    pbn = proc_bn.reshape(1, _L)
    dbe = dec_be.reshape(1, _L)
    dbn = dec_bn.reshape(1, _L)

    zero = jnp.zeros((_N, _L), _f32)

    loc1 = _edge1(edge_attr, we_e, ebe)
    a1, b1, nx = _node1(x, we_s, we_r, wn_x)
    h1, aggp1 = _sc_stage_h(sr, loc1, a1, b1, zero)

    hx, p2s, p2r, n2 = _node2(aggp1, nx, wn_a, ebn,
                              pwes1 + pwes2, pwer1 + pwer2, pwn1 + pwn2)
    loc2, partb = _edge2(h1, pwe1 + pwe2, pwe2, pbe)
    h2, aggp2 = _sc_stage_h(sr, loc2, p2s, p2r, zero)

    p3s, p3r, n3 = _node3(aggp2, n2, hx, pwna, pbn,
                          pwes1, pwes2, pwer1, pwer2, pwn1, pwn2)
    loc3 = _edge3(h2, partb, pwe1, pbe)
    h3, aggp3 = _sc_stage_h(sr, loc3, p3s, p3r, zero)

    p4s, p4r, n4 = _node4(aggp3, n3, pwna, pbn, dwe_s, dwe_r, dwn_x)
    loc4 = _edge4(h3, dwe_e, dbe)
    aggp4 = _sc_stage_last(sr, loc4, p4s, p4r, zero)

    return _node5(aggp4, n4, dwn_a, dbn)


# speedup vs baseline: 9.3515x; 2.1609x over previous
import jax
import jax.numpy as jnp
from jax import lax
from jax.experimental import pallas as pl
from jax.experimental.pallas import tpu as pltpu
from jax.experimental.pallas import tpu_sc as plsc

_N = 10000
_E = 320000
_L = 32
_E4 = _E // 4
_NC = 2
_NS = 16
_NW = _NC * _NS
_CHUNK = 128
_CP = _CHUNK // 4
_NCHUNKS = _E // _CHUNK
_RPS = (_N // _NS) // 8 * 8
_BP = 4000
_DEPTH = 6
_NJ = (_NCHUNKS // _NW) // _DEPTH * _DEPTH
_NIT = _NJ // _DEPTH
_NTAIL = _NCHUNKS - _NJ * _NW

_f32 = jnp.float32


def _make_sc_stage(write_h):
    mesh = plsc.VectorSubcoreMesh(core_axis_name="c", subcore_axis_name="s",
                                  num_cores=_NC, num_subcores=_NS)
    out_type = [jax.ShapeDtypeStruct((_NC, _N, _L), _f32)]
    if write_h:
        out_type = [jax.ShapeDtypeStruct((_E4, 128), _f32)] + out_type
    slot_scratch = [
        pltpu.VMEM((2, _CHUNK), jnp.int32),
        pltpu.VMEM((_CP, 128), _f32),
        pltpu.VMEM((_CHUNK, _L), _f32),
        pltpu.VMEM((_CHUNK, _L), _f32),
        pltpu.VMEM((_CHUNK, _L), _f32),
    ]
    scratch_types = (slot_scratch * _DEPTH
                     + [pltpu.VMEM_SHARED((_N, _L), _f32)]
                     + [pltpu.SemaphoreType.DMA] * (_DEPTH + 2))

    def body(s_hbm, r_hbm, loc_hbm, tabs_hbm, tabr_hbm, zero_hbm, *rest):
        if write_h:
            h_hbm, aggp_hbm = rest[0], rest[1]
            rest = rest[2:]
        else:
            aggp_hbm = rest[0]
            rest = rest[1:]
        slots = [rest[5 * k:5 * k + 5] for k in range(_DEPTH)]
        agg_sh = rest[5 * _DEPTH]
        semg = rest[5 * _DEPTH + 1:5 * _DEPTH + 1 + _DEPTH]
        semi = rest[5 * _DEPTH + 1 + _DEPTH]
        semw = rest[5 * _DEPTH + 2 + _DEPTH]
        cid = lax.axis_index("c")
        sid = lax.axis_index("s")
        wid = sid * _NC + cid

        @pl.when(sid == 0)
        def _():
            pltpu.sync_copy(zero_hbm, agg_sh)

        plsc.subcore_barrier()

        def compute(b):
            _, lv, gs, gr, hv = slots[b]

            @plsc.parallel_loop(0, _CP, unroll=2)
            def _(p):
                for q in range(4):
                    e = p * 4 + q
                    for half in range(2):
                        col = 32 * q + 16 * half
                        v = (lv[p, pl.ds(col, 16)]
                             + gs[e, pl.ds(16 * half, 16)]
                             + gr[e, pl.ds(16 * half, 16)])
                        v = jnp.maximum(v, 0.0)
                        hv[e, pl.ds(16 * half, 16)] = v
                        if write_h:
                            lv[p, pl.ds(col, 16)] = v

        def iteration(it, carry):
            c0 = wid + it * (_DEPTH * _NW)
            idx_waits = []
            for b in range(_DEPTH):
                base = (c0 + b * _NW) * _CHUNK
                idxv = slots[b][0]
                idx_waits.append(pltpu.async_copy(
                    s_hbm.at[pl.ds(base, _CHUNK)], idxv.at[0], semi))
                idx_waits.append(pltpu.async_copy(
                    r_hbm.at[pl.ds(base, _CHUNK)], idxv.at[1], semi))
            for d in idx_waits:
                d.wait()
            data_waits = []
            for b in range(_DEPTH):
                idxv, lv, gs, gr, hv = slots[b]
                pbase = (c0 + b * _NW) * _CP
                d1 = pltpu.async_copy(loc_hbm.at[pl.ds(pbase, _CP)], lv, semg[b])
                d2 = pltpu.async_copy(tabs_hbm.at[idxv.at[0]], gs, semg[b])
                d3 = pltpu.async_copy(tabr_hbm.at[idxv.at[1]], gr, semg[b])
                data_waits.append((d1, d2, d3))
            h_waits = []
            for b in range(_DEPTH):
                idxv, lv, gs, gr, hv = slots[b]
                for d in data_waits[b]:
                    d.wait()
                compute(b)
                if write_h:
                    pbase = (c0 + b * _NW) * _CP
                    h_waits.append(pltpu.async_copy(
                        lv, h_hbm.at[pl.ds(pbase, _CP)], semw))
                pltpu.sync_copy(hv, agg_sh.at[idxv.at[1]], add=True)
            for d in h_waits:
                d.wait()
            return carry

        lax.fori_loop(0, _NIT, iteration, 0)

        @pl.when(wid < _NTAIL)
        def _():
            ct = _NJ * _NW + wid
            base = ct * _CHUNK
            pbase = ct * _CP
            idxv, lv, gs, gr, hv = slots[0]
            pltpu.sync_copy(s_hbm.at[pl.ds(base, _CHUNK)], idxv.at[0])
            pltpu.sync_copy(r_hbm.at[pl.ds(base, _CHUNK)], idxv.at[1])
            pltpu.sync_copy(loc_hbm.at[pl.ds(pbase, _CP)], lv)
            pltpu.async_copy(tabs_hbm.at[idxv.at[0]], gs, semg[0]).wait()
            pltpu.async_copy(tabr_hbm.at[idxv.at[1]], gr, semg[0]).wait()
            compute(0)
            if write_h:
                pltpu.sync_copy(lv, h_hbm.at[pl.ds(pbase, _CP)])
            pltpu.sync_copy(hv, agg_sh.at[idxv.at[1]], add=True)

        plsc.subcore_barrier()
        pltpu.sync_copy(
            agg_sh.at[pl.ds(sid * _RPS, _RPS)],
            aggp_hbm.at[cid, pl.ds(sid * _RPS, _RPS)],
        )

        @pl.when(sid == 0)
        def _():
            pltpu.sync_copy(
                agg_sh.at[pl.ds(_NS * _RPS, _N - _NS * _RPS)],
                aggp_hbm.at[cid, pl.ds(_NS * _RPS, _N - _NS * _RPS)],
            )

    return pl.kernel(body, out_type=tuple(out_type) if write_h else out_type[0],
                     mesh=mesh, scratch_types=scratch_types,
                     compiler_params=pltpu.CompilerParams(use_tc_tiling_on_sc=False))


_sc_stage_h = _make_sc_stage(True)
_sc_stage_last = _make_sc_stage(False)


def _dot(a, b):
    return jnp.dot(a, b, preferred_element_type=_f32)


def _stage_call(nk, edge_widths, full_shapes, n_edge_out, n_node_out):
    grid = (_E4 // _BP,)
    in_specs = [pl.BlockSpec((_BP, w), lambda i: (i, 0)) for w in edge_widths]
    in_specs += [pl.BlockSpec(shp, lambda i, n=len(shp): (0,) * n)
                 for shp in full_shapes]
    out_specs = ([pl.BlockSpec((_BP, 128), lambda i: (i, 0))] * n_edge_out
                 + [pl.BlockSpec((_N, _L), lambda i: (0, 0))] * n_node_out)
    out_shape = ([jax.ShapeDtypeStruct((_E4, 128), _f32)] * n_edge_out
                 + [jax.ShapeDtypeStruct((_N, _L), _f32)] * n_node_out)
    return pl.pallas_call(nk, grid=grid, in_specs=in_specs,
                          out_specs=out_specs, out_shape=out_shape)


def _stage1_k(ea_ref, x_ref, wee_ref, ebe_ref, wes_ref, wer_ref, wnx_ref,
              loc1_ref, a_ref, b_ref, nx_ref):
    loc1_ref[...] = _dot(ea_ref[...], wee_ref[...]) + ebe_ref[...]

    @pl.when(pl.program_id(0) == 0)
    def _():
        xx = x_ref[...]
        a_ref[...] = _dot(xx, wes_ref[...])
        b_ref[...] = _dot(xx, wer_ref[...])
        nx_ref[...] = _dot(xx, wnx_ref[...])


def _stage2_k(h1_ref, aggp_ref, nx_ref, wl_ref, w2_ref, pbe_ref,
              wna_ref, ebn_ref, ws_ref, wr_ref, wn_ref,
              loc2_ref, partb_ref, hx_ref, p2s_ref, p2r_ref, n2_ref):
    h = h1_ref[...]
    loc2_ref[...] = _dot(h, wl_ref[...]) + pbe_ref[...]
    partb_ref[...] = _dot(h, w2_ref[...])

    @pl.when(pl.program_id(0) == 0)
    def _():
        agg = aggp_ref[0] + aggp_ref[1]
        hx = jnp.maximum(nx_ref[...] + _dot(agg, wna_ref[...]) + ebn_ref[...], 0.0)
        hx_ref[...] = hx
        p2s_ref[...] = _dot(hx, ws_ref[...])
        p2r_ref[...] = _dot(hx, wr_ref[...])
        n2_ref[...] = _dot(hx, wn_ref[...])


def _stage3_k(h2_ref, pb_ref, aggp_ref, n2_ref, hxin_ref, w_ref, pbe_ref,
              wna_ref, pbn_ref, ws1_ref, ws2_ref, wr1_ref, wr2_ref,
              wn1_ref, wn2_ref,
              loc3_ref, p3s_ref, p3r_ref, n3_ref):
    loc3_ref[...] = _dot(h2_ref[...], w_ref[...]) + pb_ref[...] + pbe_ref[...]

    @pl.when(pl.program_id(0) == 0)
    def _():
        agg = aggp_ref[0] + aggp_ref[1]
        cx2 = jnp.maximum(n2_ref[...] + _dot(agg, wna_ref[...]) + pbn_ref[...], 0.0)
        hx = hxin_ref[...]
        p3s_ref[...] = _dot(cx2, ws1_ref[...]) + _dot(hx, ws2_ref[...])
        p3r_ref[...] = _dot(cx2, wr1_ref[...]) + _dot(hx, wr2_ref[...])
        n3_ref[...] = _dot(cx2, wn1_ref[...]) + _dot(hx, wn2_ref[...])


def _stage4_k(h3_ref, aggp_ref, n3_ref, w_ref, dbe_ref, wna_ref, pbn_ref,
              ws_ref, wr_ref, wnx_ref,
              loc4_ref, p4s_ref, p4r_ref, n4_ref):
    loc4_ref[...] = _dot(h3_ref[...], w_ref[...]) + dbe_ref[...]

    @pl.when(pl.program_id(0) == 0)
    def _():
        agg = aggp_ref[0] + aggp_ref[1]
        cx3 = jnp.maximum(n3_ref[...] + _dot(agg, wna_ref[...]) + pbn_ref[...], 0.0)
        p4s_ref[...] = _dot(cx3, ws_ref[...])
        p4r_ref[...] = _dot(cx3, wr_ref[...])
        n4_ref[...] = _dot(cx3, wnx_ref[...])


def _node5_k(aggp_ref, n4_ref, wna_ref, bn_ref, o_ref):
    agg = aggp_ref[0] + aggp_ref[1]
    o_ref[...] = jnp.maximum(n4_ref[...] + _dot(agg, wna_ref[...]) + bn_ref[...], 0.0)


_WL = (_L, _L)
_W44 = (128, 128)
_B4 = (1, 128)
_NP = (_NC, _N, _L)
_stage1 = _stage_call(_stage1_k, [64],
                      [(_N, 128), (64, 128), _B4, (128, _L), (128, _L), (128, _L)],
                      1, 3)
_stage2 = _stage_call(_stage2_k, [128],
                      [_NP, (_N, _L), _W44, _W44, _B4, _WL, (1, _L), _WL, _WL, _WL],
                      2, 4)
_stage3 = _stage_call(_stage3_k, [128, 128],
                      [_NP, (_N, _L), (_N, _L), _W44, _B4, _WL, (1, _L),
                       _WL, _WL, _WL, _WL, _WL, _WL],
                      1, 3)
_stage4 = _stage_call(_stage4_k, [128],
                      [_NP, (_N, _L), _W44, _B4, _WL, (1, _L), _WL, _WL, _WL],
                      1, 3)
_node5 = pl.pallas_call(_node5_k, out_shape=jax.ShapeDtypeStruct((_N, _L), _f32))


def kernel(x, edge_attr, edge_index,
           enc_We, enc_be, enc_Wn, enc_bn,
           proc_We, proc_be, proc_Wn, proc_bn,
           dec_We, dec_be, dec_Wn, dec_bn):
    s = edge_index[0]
    r = edge_index[1]
    ea4 = edge_attr.reshape(_E4, 64)

    we_e, we_s, we_r = enc_We[:16], enc_We[16:144], enc_We[144:272]
    wn_x, wn_a = enc_Wn[:128], enc_Wn[128:160]
    pwe1, pwe2 = proc_We[0:32], proc_We[32:64]
    pwes1, pwes2 = proc_We[64:96], proc_We[96:128]
    pwer1, pwer2 = proc_We[128:160], proc_We[160:192]
    pwn1, pwn2, pwna = proc_Wn[0:32], proc_Wn[32:64], proc_Wn[64:96]
    dwe_e, dwe_s, dwe_r = dec_We[0:32], dec_We[32:64], dec_We[64:96]
    dwn_x, dwn_a = dec_Wn[0:32], dec_Wn[32:64]

    eye4 = jnp.eye(4, dtype=_f32)
    kron = jnp.kron
    wee4 = kron(eye4, we_e)
    wl4 = kron(eye4, pwe1 + pwe2)
    w24 = kron(eye4, pwe2)
    pwe14 = kron(eye4, pwe1)
    dwee4 = kron(eye4, dwe_e)

    ebe4 = jnp.tile(enc_be.reshape(1, _L), (1, 4))
    pbe4 = jnp.tile(proc_be.reshape(1, _L), (1, 4))
    dbe4 = jnp.tile(dec_be.reshape(1, _L), (1, 4))
    ebn = enc_bn.reshape(1, _L)
    pbn = proc_bn.reshape(1, _L)
    dbn = dec_bn.reshape(1, _L)

    zero = jnp.zeros((_N, _L), _f32)

    loc1, a1, b1, nx = _stage1(ea4, x, wee4, ebe4, we_s, we_r, wn_x)
    h1, aggp1 = _sc_stage_h(s, r, loc1, a1, b1, zero)

    loc2, partb, hx, p2s, p2r, n2 = _stage2(
        h1, aggp1, nx, wl4, w24, pbe4, wn_a, ebn,
        pwes1 + pwes2, pwer1 + pwer2, pwn1 + pwn2)
    h2, aggp2 = _sc_stage_h(s, r, loc2, p2s, p2r, zero)

    loc3, p3s, p3r, n3 = _stage3(
        h2, partb, aggp2, n2, hx, pwe14, pbe4, pwna, pbn,
        pwes1, pwes2, pwer1, pwer2, pwn1, pwn2)
    h3, aggp3 = _sc_stage_h(s, r, loc3, p3s, p3r, zero)

    loc4, p4s, p4r, n4 = _stage4(
        h3, aggp3, n3, dwee4, dbe4, pwna, pbn, dwe_s, dwe_r, dwn_x)
    aggp4 = _sc_stage_last(s, r, loc4, p4s, p4r, zero)

    return _node5(aggp4, n4, dwn_a, dbn)


# speedup vs baseline: 9.7512x; 1.0427x over previous
import jax
import jax.numpy as jnp
from jax import lax
from jax.experimental import pallas as pl
from jax.experimental.pallas import tpu as pltpu
from jax.experimental.pallas import tpu_sc as plsc

_N = 10000
_E = 320000
_L = 32
_E4 = _E // 4
_NC = 2
_NS = 16
_NW = _NC * _NS
_CHUNK = 128
_CP = _CHUNK // 4
_NCHUNKS = _E // _CHUNK
_RPS = (_N // _NS) // 8 * 8
_BP = 4000
_DEPTH = 6
_NJ = (_NCHUNKS // _NW) // _DEPTH * _DEPTH
_NIT = _NJ // _DEPTH
_NTAIL = _NCHUNKS - _NJ * _NW

_f32 = jnp.float32


def _make_sc_stage(write_h):
    mesh = plsc.VectorSubcoreMesh(core_axis_name="c", subcore_axis_name="s",
                                  num_cores=_NC, num_subcores=_NS)
    out_type = [jax.ShapeDtypeStruct((_NC, _N, _L), _f32)]
    if write_h:
        out_type = [jax.ShapeDtypeStruct((_E4, 128), _f32)] + out_type
    scratch_types = ([pltpu.VMEM((_DEPTH, _CHUNK), jnp.int32)] * 2
                     + [pltpu.VMEM((_DEPTH * _CP, 128), _f32)]
                     + [pltpu.VMEM((_CHUNK, _L), _f32)] * (3 * _DEPTH)
                     + [pltpu.VMEM_SHARED((_N, _L), _f32)]
                     + [pltpu.SemaphoreType.DMA] * (_DEPTH + 2))

    def body(s_hbm, r_hbm, loc_hbm, tabs_hbm, tabr_hbm, zero_hbm, *rest):
        if write_h:
            h_hbm, aggp_hbm = rest[0], rest[1]
            rest = rest[2:]
        else:
            aggp_hbm = rest[0]
            rest = rest[1:]
        sb, rb, lvb = rest[0], rest[1], rest[2]
        gsl = rest[3:3 + _DEPTH]
        grl = rest[3 + _DEPTH:3 + 2 * _DEPTH]
        hvl = rest[3 + 2 * _DEPTH:3 + 3 * _DEPTH]
        agg_sh = rest[3 + 3 * _DEPTH]
        semg = rest[4 + 3 * _DEPTH:4 + 4 * _DEPTH]
        semi = rest[4 + 4 * _DEPTH]
        semw = rest[5 + 4 * _DEPTH]
        cid = lax.axis_index("c")
        sid = lax.axis_index("s")
        wid = sid * _NC + cid

        @pl.when(sid == 0)
        def _():
            pltpu.sync_copy(zero_hbm, agg_sh)

        plsc.subcore_barrier()

        def compute(b, roff):
            gs, gr, hv = gsl[b], grl[b], hvl[b]

            @plsc.parallel_loop(0, _CP, unroll=2)
            def _(p):
                row = roff + p
                for q in range(4):
                    e = p * 4 + q
                    for half in range(2):
                        col = 32 * q + 16 * half
                        v = (lvb[row, pl.ds(col, 16)]
                             + gs[e, pl.ds(16 * half, 16)]
                             + gr[e, pl.ds(16 * half, 16)])
                        v = jnp.maximum(v, 0.0)
                        hv[e, pl.ds(16 * half, 16)] = v
                        if write_h:
                            lvb[row, pl.ds(col, 16)] = v

        def iteration(it, carry):
            c0 = _NJ * wid + it * _DEPTH
            pbase = c0 * _CP
            di1 = pltpu.async_copy(s_hbm.at[pl.ds(c0, _DEPTH)], sb, semi)
            di2 = pltpu.async_copy(r_hbm.at[pl.ds(c0, _DEPTH)], rb, semi)
            dloc = pltpu.async_copy(
                loc_hbm.at[pl.ds(pbase, _DEPTH * _CP)], lvb, semw)
            di1.wait()
            di2.wait()
            data_waits = []
            for b in range(_DEPTH):
                d2 = pltpu.async_copy(tabs_hbm.at[sb.at[b]], gsl[b], semg[b])
                d3 = pltpu.async_copy(tabr_hbm.at[rb.at[b]], grl[b], semg[b])
                data_waits.append((d2, d3))
            dloc.wait()
            for b in range(_DEPTH):
                for d in data_waits[b]:
                    d.wait()
                compute(b, b * _CP)
            if write_h:
                dh = pltpu.async_copy(
                    lvb, h_hbm.at[pl.ds(pbase, _DEPTH * _CP)], semw)
            for b in range(_DEPTH):
                pltpu.sync_copy(hvl[b], agg_sh.at[rb.at[b]], add=True)
            if write_h:
                dh.wait()
            return carry

        lax.fori_loop(0, _NIT, iteration, 0)

        @pl.when(wid < _NTAIL)
        def _():
            ct = _NJ * _NW + wid
            pbase = ct * _CP
            lv = lvb.at[pl.ds(0, _CP)]
            pltpu.sync_copy(s_hbm.at[pl.ds(ct, 1)], sb.at[pl.ds(0, 1)])
            pltpu.sync_copy(r_hbm.at[pl.ds(ct, 1)], rb.at[pl.ds(0, 1)])
            pltpu.sync_copy(loc_hbm.at[pl.ds(pbase, _CP)], lv)
            pltpu.async_copy(tabs_hbm.at[sb.at[0]], gsl[0], semg[0]).wait()
            pltpu.async_copy(tabr_hbm.at[rb.at[0]], grl[0], semg[0]).wait()
            compute(0, 0)
            if write_h:
                pltpu.sync_copy(lv, h_hbm.at[pl.ds(pbase, _CP)])
            pltpu.sync_copy(hvl[0], agg_sh.at[rb.at[0]], add=True)

        plsc.subcore_barrier()
        pltpu.sync_copy(
            agg_sh.at[pl.ds(sid * _RPS, _RPS)],
            aggp_hbm.at[cid, pl.ds(sid * _RPS, _RPS)],
        )

        @pl.when(sid == 0)
        def _():
            pltpu.sync_copy(
                agg_sh.at[pl.ds(_NS * _RPS, _N - _NS * _RPS)],
                aggp_hbm.at[cid, pl.ds(_NS * _RPS, _N - _NS * _RPS)],
            )

    return pl.kernel(body, out_type=tuple(out_type) if write_h else out_type[0],
                     mesh=mesh, scratch_types=scratch_types,
                     compiler_params=pltpu.CompilerParams(use_tc_tiling_on_sc=False))


_sc_stage_h = _make_sc_stage(True)
_sc_stage_last = _make_sc_stage(False)


def _dot(a, b):
    return jnp.dot(a, b, preferred_element_type=_f32)


def _stage_call(nk, edge_widths, full_shapes, n_edge_out, n_node_out):
    grid = (_E4 // _BP,)
    in_specs = [pl.BlockSpec((_BP, w), lambda i: (i, 0)) for w in edge_widths]
    in_specs += [pl.BlockSpec(shp, lambda i, n=len(shp): (0,) * n)
                 for shp in full_shapes]
    out_specs = ([pl.BlockSpec((_BP, 128), lambda i: (i, 0))] * n_edge_out
                 + [pl.BlockSpec((_N, _L), lambda i: (0, 0))] * n_node_out)
    out_shape = ([jax.ShapeDtypeStruct((_E4, 128), _f32)] * n_edge_out
                 + [jax.ShapeDtypeStruct((_N, _L), _f32)] * n_node_out)
    return pl.pallas_call(nk, grid=grid, in_specs=in_specs,
                          out_specs=out_specs, out_shape=out_shape)


def _stage1_k(ea_ref, x_ref, wee_ref, ebe_ref, wes_ref, wer_ref, wnx_ref,
              loc1_ref, a_ref, b_ref, nx_ref):
    loc1_ref[...] = _dot(ea_ref[...], wee_ref[...]) + ebe_ref[...]

    @pl.when(pl.program_id(0) == 0)
    def _():
        xx = x_ref[...]
        a_ref[...] = _dot(xx, wes_ref[...])
        b_ref[...] = _dot(xx, wer_ref[...])
        nx_ref[...] = _dot(xx, wnx_ref[...])


def _stage2_k(h1_ref, aggp_ref, nx_ref, wl_ref, w2_ref, pbe_ref,
              wna_ref, ebn_ref, ws_ref, wr_ref, wn_ref,
              loc2_ref, partb_ref, hx_ref, p2s_ref, p2r_ref, n2_ref):
    h = h1_ref[...]
    loc2_ref[...] = _dot(h, wl_ref[...]) + pbe_ref[...]
    partb_ref[...] = _dot(h, w2_ref[...])

    @pl.when(pl.program_id(0) == 0)
    def _():
        agg = aggp_ref[0] + aggp_ref[1]
        hx = jnp.maximum(nx_ref[...] + _dot(agg, wna_ref[...]) + ebn_ref[...], 0.0)
        hx_ref[...] = hx
        p2s_ref[...] = _dot(hx, ws_ref[...])
        p2r_ref[...] = _dot(hx, wr_ref[...])
        n2_ref[...] = _dot(hx, wn_ref[...])


def _stage3_k(h2_ref, pb_ref, aggp_ref, n2_ref, hxin_ref, w_ref, pbe_ref,
              wna_ref, pbn_ref, ws1_ref, ws2_ref, wr1_ref, wr2_ref,
              wn1_ref, wn2_ref,
              loc3_ref, p3s_ref, p3r_ref, n3_ref):
    loc3_ref[...] = _dot(h2_ref[...], w_ref[...]) + pb_ref[...] + pbe_ref[...]

    @pl.when(pl.program_id(0) == 0)
    def _():
        agg = aggp_ref[0] + aggp_ref[1]
        cx2 = jnp.maximum(n2_ref[...] + _dot(agg, wna_ref[...]) + pbn_ref[...], 0.0)
        hx = hxin_ref[...]
        p3s_ref[...] = _dot(cx2, ws1_ref[...]) + _dot(hx, ws2_ref[...])
        p3r_ref[...] = _dot(cx2, wr1_ref[...]) + _dot(hx, wr2_ref[...])
        n3_ref[...] = _dot(cx2, wn1_ref[...]) + _dot(hx, wn2_ref[...])


def _stage4_k(h3_ref, aggp_ref, n3_ref, w_ref, dbe_ref, wna_ref, pbn_ref,
              ws_ref, wr_ref, wnx_ref,
              loc4_ref, p4s_ref, p4r_ref, n4_ref):
    loc4_ref[...] = _dot(h3_ref[...], w_ref[...]) + dbe_ref[...]

    @pl.when(pl.program_id(0) == 0)
    def _():
        agg = aggp_ref[0] + aggp_ref[1]
        cx3 = jnp.maximum(n3_ref[...] + _dot(agg, wna_ref[...]) + pbn_ref[...], 0.0)
        p4s_ref[...] = _dot(cx3, ws_ref[...])
        p4r_ref[...] = _dot(cx3, wr_ref[...])
        n4_ref[...] = _dot(cx3, wnx_ref[...])


def _node5_k(aggp_ref, n4_ref, wna_ref, bn_ref, o_ref):
    agg = aggp_ref[0] + aggp_ref[1]
    o_ref[...] = jnp.maximum(n4_ref[...] + _dot(agg, wna_ref[...]) + bn_ref[...], 0.0)


_WL = (_L, _L)
_W44 = (128, 128)
_B4 = (1, 128)
_NP = (_NC, _N, _L)
_stage1 = _stage_call(_stage1_k, [64],
                      [(_N, 128), (64, 128), _B4, (128, _L), (128, _L), (128, _L)],
                      1, 3)
_stage2 = _stage_call(_stage2_k, [128],
                      [_NP, (_N, _L), _W44, _W44, _B4, _WL, (1, _L), _WL, _WL, _WL],
                      2, 4)
_stage3 = _stage_call(_stage3_k, [128, 128],
                      [_NP, (_N, _L), (_N, _L), _W44, _B4, _WL, (1, _L),
                       _WL, _WL, _WL, _WL, _WL, _WL],
                      1, 3)
_stage4 = _stage_call(_stage4_k, [128],
                      [_NP, (_N, _L), _W44, _B4, _WL, (1, _L), _WL, _WL, _WL],
                      1, 3)
_node5 = pl.pallas_call(_node5_k, out_shape=jax.ShapeDtypeStruct((_N, _L), _f32))


def kernel(x, edge_attr, edge_index,
           enc_We, enc_be, enc_Wn, enc_bn,
           proc_We, proc_be, proc_Wn, proc_bn,
           dec_We, dec_be, dec_Wn, dec_bn):
    s = edge_index[0].reshape(_NCHUNKS, _CHUNK)
    r = edge_index[1].reshape(_NCHUNKS, _CHUNK)
    ea4 = edge_attr.reshape(_E4, 64)

    we_e, we_s, we_r = enc_We[:16], enc_We[16:144], enc_We[144:272]
    wn_x, wn_a = enc_Wn[:128], enc_Wn[128:160]
    pwe1, pwe2 = proc_We[0:32], proc_We[32:64]
    pwes1, pwes2 = proc_We[64:96], proc_We[96:128]
    pwer1, pwer2 = proc_We[128:160], proc_We[160:192]
    pwn1, pwn2, pwna = proc_Wn[0:32], proc_Wn[32:64], proc_Wn[64:96]
    dwe_e, dwe_s, dwe_r = dec_We[0:32], dec_We[32:64], dec_We[64:96]
    dwn_x, dwn_a = dec_Wn[0:32], dec_Wn[32:64]

    eye4 = jnp.eye(4, dtype=_f32)
    kron = jnp.kron
    wee4 = kron(eye4, we_e)
    wl4 = kron(eye4, pwe1 + pwe2)
    w24 = kron(eye4, pwe2)
    pwe14 = kron(eye4, pwe1)
    dwee4 = kron(eye4, dwe_e)

    ebe4 = jnp.tile(enc_be.reshape(1, _L), (1, 4))
    pbe4 = jnp.tile(proc_be.reshape(1, _L), (1, 4))
    dbe4 = jnp.tile(dec_be.reshape(1, _L), (1, 4))
    ebn = enc_bn.reshape(1, _L)
    pbn = proc_bn.reshape(1, _L)
    dbn = dec_bn.reshape(1, _L)

    zero = jnp.zeros((_N, _L), _f32)

    loc1, a1, b1, nx = _stage1(ea4, x, wee4, ebe4, we_s, we_r, wn_x)
    h1, aggp1 = _sc_stage_h(s, r, loc1, a1, b1, zero)

    loc2, partb, hx, p2s, p2r, n2 = _stage2(
        h1, aggp1, nx, wl4, w24, pbe4, wn_a, ebn,
        pwes1 + pwes2, pwer1 + pwer2, pwn1 + pwn2)
    h2, aggp2 = _sc_stage_h(s, r, loc2, p2s, p2r, zero)

    loc3, p3s, p3r, n3 = _stage3(
        h2, partb, aggp2, n2, hx, pwe14, pbe4, pwna, pbn,
        pwes1, pwes2, pwer1, pwer2, pwn1, pwn2)
    h3, aggp3 = _sc_stage_h(s, r, loc3, p3s, p3r, zero)

    loc4, p4s, p4r, n4 = _stage4(
        h3, aggp3, n3, dwee4, dbe4, pwna, pbn, dwe_s, dwe_r, dwn_x)
    aggp4 = _sc_stage_last(s, r, loc4, p4s, p4r, zero)

    return _node5(aggp4, n4, dwn_a, dbn)
